# baseline (speedup 1.0000x reference)
import functools

import jax
import jax.numpy as jnp
from jax import lax
from jax.experimental import pallas as pl
from jax.experimental.pallas import tpu as pltpu
from jax.experimental.pallas import tpu_sc as plsc

V = 100000
D = 64
WIN = 20
DH = 128
B = 1024

TV = 5120
NVB = (V + TV - 1) // TV

N = B * WIN
NPAD = N + 16
DUMMY = N
TCW = 128
TPW = 25
WRANGE = TPW * TCW
TAIL_LO = V - 32


def _count(m):
    return jnp.sum(m.astype(jnp.int32))


def _sc_gather(embT, idx_flat, tail_rows):
    info = plsc.get_sparse_core_info()
    nc_ = info.num_cores
    mesh = plsc.VectorSubcoreMesh(core_axis_name="c", subcore_axis_name="s")

    @functools.partial(
        pl.kernel,
        mesh=mesh,
        out_type=jax.ShapeDtypeStruct((N + 1, D), jnp.float32),
        compiler_params=pltpu.CompilerParams(
            use_tc_tiling_on_sc=False, needs_layout_passes=False),
        scratch_types=[
            pltpu.VMEM((NPAD,), jnp.int32),
            pltpu.VMEM((NPAD,), jnp.int32),
            pltpu.VMEM((N + 64,), jnp.int32),
            pltpu.VMEM((D, TCW), jnp.float32),
            pltpu.VMEM((32, D), jnp.float32),
            pltpu.VMEM((64, D), jnp.float32),
            pltpu.VMEM((64,), jnp.int32),
            pltpu.SemaphoreType.DMA,
        ],
    )
    def tg_kernel(embT_hbm, idx_hbm, tail_hbm, out_hbm,
                  idx_v, pos_v, cpos_v, slab_v, tail_v, outb_v, gidx_v, sem):
        wid = lax.axis_index("s") * nc_ + lax.axis_index("c")
        iota = lax.iota(jnp.int32, 16)
        dvec = jnp.full((16,), DUMMY, jnp.int32)

        def prefix_incl(s):
            for k in (1, 2, 4, 8):
                idxk = jnp.maximum(iota - k, 0)
                g = s.at[idxk].get(mode="promise_in_bounds")
                s = s + jnp.where(iota >= k, g, 0)
            return s

        pltpu.sync_copy(idx_hbm, idx_v.at[pl.ds(0, N)])

        def pre_body(i, carry):
            pos_v[pl.ds(i * 16, 16)] = dvec
            return carry
        lax.fori_loop(0, NPAD // 16, pre_body, jnp.int32(0))

        def scan_body(i, nw):
            v = idx_v[pl.ds(i * 16, 16)]
            m = (((v >> 7) * 5243) >> 17) == wid
            s = prefix_incl(m.astype(jnp.int32))
            dst = jnp.where(m, nw + s - 1, NPAD - 1)
            plsc.store_scatter(pos_v, [dst], iota + i * 16)
            return nw + jnp.sum(m.astype(jnp.int32))
        n_w = lax.fori_loop(0, N // 16, scan_body, jnp.int32(0))
        ntrips = (n_w + 15) // 16

        def chunk_process(lo, width, gather_vals):
            def cscan(i, ncc):
                p = pos_v[pl.ds(i * 16, 16)]
                v = plsc.load_gather(idx_v, [p])
                colr = v - lo
                m = (colr >= 0) & (colr < width)
                s = prefix_incl(m.astype(jnp.int32))
                dst = jnp.where(m, ncc + s - 1, N + 63)
                plsc.store_scatter(cpos_v, [dst], p)
                return ncc + jnp.sum(m.astype(jnp.int32))
            ncr = lax.fori_loop(0, ntrips, cscan, jnp.int32(0))
            fb = (ncr // 16) * 16
            for g in range(4):
                off = fb + g * 16
                cur = cpos_v[pl.ds(off, 16)]
                cpos_v[pl.ds(off, 16)] = jnp.where(off + iota < ncr,
                                                   cur, dvec)
            def blk(b, carry):
                for sb in range(4):
                    p16 = cpos_v[pl.ds(b * 64 + sb * 16, 16)]
                    v16 = plsc.load_gather(idx_v, [p16])
                    j = jnp.minimum(jnp.maximum(v16 - lo, 0), width - 1)
                    rows = iota + sb * 16
                    for d in range(D):
                        dd = jnp.full((16,), d, jnp.int32)
                        vals = gather_vals(j, dd)
                        plsc.store_scatter(outb_v, [rows, dd], vals)
                for q in range(4):
                    gidx_v[pl.ds(q * 16, 16)] = cpos_v[pl.ds(b * 64 + q * 16,
                                                             16)]
                pltpu.async_copy(outb_v, out_hbm.at[gidx_v], sem).wait()
                return carry
            lax.fori_loop(0, (ncr + 63) // 64, blk, jnp.int32(0))

        nchunks = jnp.where(wid == 31, jnp.int32(6), jnp.int32(TPW))

        def chunk_body(c, carry):
            lo = pl.multiple_of((wid * TPW + c) * TCW, TCW)
            pltpu.sync_copy(embT_hbm.at[:, pl.ds(lo, TCW)], slab_v)
            chunk_process(
                lo, TCW,
                lambda j, dd: plsc.load_gather(slab_v, [dd, j]))
            return carry
        lax.fori_loop(0, nchunks, chunk_body, jnp.int32(0))

        @pl.when(wid == 31)
        def _():
            pltpu.sync_copy(tail_hbm, tail_v)
            chunk_process(
                jnp.int32(TAIL_LO), 32,
                lambda j, dd: plsc.load_gather(tail_v, [j, dd]))

    return tg_kernel(embT, idx_flat, tail_rows)


def _mlp_body(e_ref, w1_ref, b1_ref, w2_ref, out_ref, h_ref):
    @pl.when(pl.program_id(0) == 0)
    def _():
        h = lax.dot_general(
            e_ref[...], w1_ref[...], (((1,), (1,)), ((), ())),
            preferred_element_type=jnp.float32)
        h_ref[...] = jnp.maximum(h + b1_ref[...], 0.0)

    out_ref[...] = lax.dot_general(
        w2_ref[...], h_ref[...], (((1,), (1,)), ((), ())),
        preferred_element_type=jnp.float32)


def _mlp(e_flat, W1, b1, W2):
    return pl.pallas_call(
        _mlp_body,
        grid=(NVB,),
        in_specs=[
            pl.BlockSpec((B, WIN * D), lambda j: (0, 0)),
            pl.BlockSpec((DH, WIN * D), lambda j: (0, 0)),
            pl.BlockSpec((1, DH), lambda j: (0, 0)),
            pl.BlockSpec((TV, DH), lambda j: (j, 0)),
        ],
        out_specs=pl.BlockSpec((TV, B), lambda j: (j, 0)),
        out_shape=jax.ShapeDtypeStruct((V, B), jnp.float32),
        scratch_shapes=[pltpu.VMEM((B, DH), jnp.float32)],
        compiler_params=pltpu.CompilerParams(
            dimension_semantics=("arbitrary",)),
    )(e_flat, W1, b1, W2)


def kernel(x, emb, W1, b1, W2):
    idx_flat = x.reshape(-1).astype(jnp.int32)
    embT = emb.T
    tail_rows = emb[TAIL_LO:, :]
    rows = _sc_gather(embT, idx_flat, tail_rows)
    e_flat = rows[:N].reshape(B, WIN * D)
    b1_2d = b1.reshape(1, DH)
    return _mlp(e_flat, W1, b1_2d, W2).T


# speedup vs baseline: 3.7167x; 3.7167x over previous
import functools

import jax
import jax.numpy as jnp
from jax import lax
from jax.experimental import pallas as pl
from jax.experimental.pallas import tpu as pltpu
from jax.experimental.pallas import tpu_sc as plsc

V = 100000
D = 64
WIN = 20
DH = 128
B = 1024

TV = 5120
NVB = (V + TV - 1) // TV


def _sc_gather(emb, idx_flat):
    info = plsc.get_sparse_core_info()
    nw = info.num_cores * info.num_subcores
    n = idx_flat.shape[0]
    b_per_w = n // nw
    mesh = plsc.VectorSubcoreMesh(core_axis_name="c", subcore_axis_name="s")

    @functools.partial(
        pl.kernel,
        mesh=mesh,
        out_type=jax.ShapeDtypeStruct((n, D), jnp.float32),
        compiler_params=pltpu.CompilerParams(use_tc_tiling_on_sc=False),
        scratch_types=[
            pltpu.VMEM((b_per_w,), jnp.int32),
            pltpu.VMEM((b_per_w, D), jnp.float32),
            pltpu.SemaphoreType.DMA,
        ],
    )
    def gather_kernel(table_hbm, idx_hbm, out_hbm, idx_v, rows_v, sem):
        wid = lax.axis_index("s") * info.num_cores + lax.axis_index("c")
        base = wid * b_per_w
        pltpu.sync_copy(idx_hbm.at[pl.ds(base, b_per_w)], idx_v)
        pltpu.async_copy(table_hbm.at[idx_v], rows_v, sem).wait()
        pltpu.sync_copy(rows_v, out_hbm.at[pl.ds(base, b_per_w)])

    return gather_kernel(emb, idx_flat)


def _mlp_body(e_ref, w1_ref, b1_ref, w2_ref, out_ref, h_ref):
    @pl.when(pl.program_id(0) == 0)
    def _():
        h = lax.dot_general(
            e_ref[...], w1_ref[...], (((1,), (1,)), ((), ())),
            preferred_element_type=jnp.float32)
        h_ref[...] = jnp.maximum(h + b1_ref[...], 0.0)

    out_ref[...] = lax.dot_general(
        w2_ref[...], h_ref[...], (((1,), (1,)), ((), ())),
        preferred_element_type=jnp.float32)


def _mlp(e_flat, W1, b1, W2):
    return pl.pallas_call(
        _mlp_body,
        grid=(NVB,),
        in_specs=[
            pl.BlockSpec((B, WIN * D), lambda j: (0, 0)),
            pl.BlockSpec((DH, WIN * D), lambda j: (0, 0)),
            pl.BlockSpec((1, DH), lambda j: (0, 0)),
            pl.BlockSpec((TV, DH), lambda j: (j, 0)),
        ],
        out_specs=pl.BlockSpec((TV, B), lambda j: (j, 0)),
        out_shape=jax.ShapeDtypeStruct((V, B), jnp.float32),
        scratch_shapes=[pltpu.VMEM((B, DH), jnp.float32)],
        compiler_params=pltpu.CompilerParams(
            dimension_semantics=("arbitrary",)),
    )(e_flat, W1, b1, W2)


def kernel(x, emb, W1, b1, W2):
    idx_flat = x.reshape(-1).astype(jnp.int32)
    rows = _sc_gather(emb, idx_flat)
    e_flat = rows.reshape(B, WIN * D)
    b1_2d = b1.reshape(1, DH)
    return _mlp(e_flat, W1, b1_2d, W2).T
